# Initial kernel scaffold; baseline (speedup 1.0000x reference)
import jax
import jax.numpy as jnp
from jax.experimental import pallas as pl


def kernel(x, edge_index, W1, b1, W2, b2):
    raise NotImplementedError("write your pallas kernel here")


# speedup vs baseline: 9.0156x; 9.0156x over previous
import functools

import jax
import jax.numpy as jnp
from jax import lax
from jax.experimental import pallas as pl
from jax.experimental.pallas import tpu as pltpu
from jax.experimental.pallas import tpu_sc as plsc

N_NODES = 10000
N_PAD = 10240
D = 128
E = 320000
NC, NS = 2, 16
NW = NC * NS
CHUNK = 128
N_CHUNKS = 80
E_PAD = NW * N_CHUNKS * CHUNK
SLAB = 20
NSLAB = N_CHUNKS // SLAB
DUMMY = N_NODES
ROWS_PER_TILE = N_PAD // NS
BLK = 512
GRID = N_PAD // BLK

_mesh = plsc.VectorSubcoreMesh(
    core_axis_name="c", subcore_axis_name="s", num_cores=NC, num_subcores=NS)


@functools.partial(
    pl.kernel,
    out_type=jax.ShapeDtypeStruct((NC, N_PAD, 16), jnp.float32),
    mesh=_mesh,
    scratch_types=[
        pltpu.VMEM((SLAB, CHUNK), jnp.int32),
        pltpu.VMEM((CHUNK,), jnp.int32),
        pltpu.VMEM((CHUNK, 16), jnp.float32),
        pltpu.VMEM((CHUNK, 16), jnp.float32),
        pltpu.VMEM_SHARED((N_PAD, 16), jnp.float32),
    ],
)
def _deg_kernel(dst_hbm, out_hbm, dst_v, dst_cur, ones_v, zeros_v, acc):
    c = lax.axis_index("c")
    s = lax.axis_index("s")
    wid = s * NC + c

    one16 = jnp.ones((16,), jnp.float32)
    zero16 = jnp.zeros((16,), jnp.float32)

    def _fill(r, _):
        ones_v[r, pl.ds(0, 16)] = one16
        zeros_v[r, pl.ds(0, 16)] = zero16
        return 0

    lax.fori_loop(0, CHUNK, _fill, 0)
    for k in range(ROWS_PER_TILE // CHUNK):
        pltpu.sync_copy(zeros_v, acc.at[pl.ds(s * ROWS_PER_TILE + k * CHUNK, CHUNK)])
    plsc.subcore_barrier()

    def _slab(t, _):
        pltpu.sync_copy(dst_hbm.at[wid * NSLAB + t], dst_v)

        def _body(j, _):
            for k in range(CHUNK // 16):
                dst_cur[pl.ds(k * 16, 16)] = dst_v[j, pl.ds(k * 16, 16)]
            pltpu.sync_copy(ones_v, acc.at[dst_cur], add=True)
            return 0

        lax.fori_loop(0, SLAB, _body, 0)
        return 0

    lax.fori_loop(0, NSLAB, _slab, 0)
    plsc.subcore_barrier()
    pltpu.sync_copy(acc.at[pl.ds(s * ROWS_PER_TILE, ROWS_PER_TILE)],
                    out_hbm.at[c, pl.ds(s * ROWS_PER_TILE, ROWS_PER_TILE)])


@functools.partial(
    pl.kernel,
    out_type=jax.ShapeDtypeStruct((NC, N_PAD, D), jnp.float32),
    mesh=_mesh,
    scratch_types=[
        pltpu.VMEM((SLAB, CHUNK), jnp.int32),
        pltpu.VMEM((SLAB, CHUNK), jnp.int32),
        pltpu.VMEM((CHUNK,), jnp.int32),
        pltpu.VMEM((CHUNK,), jnp.int32),
        pltpu.VMEM((CHUNK,), jnp.int32),
        pltpu.VMEM((CHUNK, D), jnp.float32),
        pltpu.VMEM((CHUNK, D), jnp.float32),
        pltpu.VMEM_SHARED((N_PAD, D), jnp.float32),
        pltpu.SemaphoreType.DMA,
        pltpu.SemaphoreType.DMA,
    ],
)
def _prop_kernel(y_hbm, src_hbm, dst_hbm, out_hbm,
                 src_v, dst_v, sidx0, sidx1, didx, buf0, buf1, acc, sem0, sem1):
    c = lax.axis_index("c")
    s = lax.axis_index("s")
    wid = s * NC + c

    zero16 = jnp.zeros((16,), jnp.float32)

    def _zrow(r, _):
        for k in range(D // 16):
            buf0[r, pl.ds(k * 16, 16)] = zero16
        return 0

    lax.fori_loop(0, CHUNK, _zrow, 0)
    for k in range(ROWS_PER_TILE // CHUNK):
        pltpu.sync_copy(buf0, acc.at[pl.ds(s * ROWS_PER_TILE + k * CHUNK, CHUNK)])
    plsc.subcore_barrier()

    def _fill(cur, row_ref, j):
        for k in range(CHUNK // 16):
            cur[pl.ds(k * 16, 16)] = row_ref[j, pl.ds(k * 16, 16)]

    pltpu.sync_copy(src_hbm.at[wid * NSLAB], src_v)
    pltpu.sync_copy(dst_hbm.at[wid * NSLAB], dst_v)
    _fill(sidx0, src_v, 0)
    pltpu.async_copy(y_hbm.at[sidx0], buf0, sem0)

    def _slab(t, _):
        def _body(i, _):
            j = 2 * i
            pltpu.make_async_copy(y_hbm.at[sidx0], buf0, sem0).wait()
            _fill(sidx1, src_v, j + 1)
            pltpu.async_copy(y_hbm.at[sidx1], buf1, sem1)
            _fill(didx, dst_v, j)
            pltpu.sync_copy(buf0, acc.at[didx], add=True)
            pltpu.make_async_copy(y_hbm.at[sidx1], buf1, sem1).wait()

            @pl.when(j + 2 < SLAB)
            def _():
                _fill(sidx0, src_v, j + 2)
                pltpu.async_copy(y_hbm.at[sidx0], buf0, sem0)

            _fill(didx, dst_v, j + 1)
            pltpu.sync_copy(buf1, acc.at[didx], add=True)
            return 0

        lax.fori_loop(0, SLAB // 2, _body, 0)

        @pl.when(t + 1 < NSLAB)
        def _():
            pltpu.sync_copy(src_hbm.at[wid * NSLAB + t + 1], src_v)
            pltpu.sync_copy(dst_hbm.at[wid * NSLAB + t + 1], dst_v)
            _fill(sidx0, src_v, 0)
            pltpu.async_copy(y_hbm.at[sidx0], buf0, sem0)

        return 0

    lax.fori_loop(0, NSLAB, _slab, 0)
    plsc.subcore_barrier()
    pltpu.sync_copy(acc.at[pl.ds(s * ROWS_PER_TILE, ROWS_PER_TILE)],
                    out_hbm.at[c, pl.ds(s * ROWS_PER_TILE, ROWS_PER_TILE)])


def _mm_body(x_ref, w_ref, o_ref):
    o_ref[...] = jnp.dot(x_ref[...], w_ref[...],
                         preferred_element_type=jnp.float32)


def _matmul(x, w):
    return pl.pallas_call(
        _mm_body,
        grid=(GRID,),
        in_specs=[pl.BlockSpec((BLK, D), lambda i: (i, 0)),
                  pl.BlockSpec((D, D), lambda i: (0, 0))],
        out_specs=pl.BlockSpec((BLK, D), lambda i: (i, 0)),
        out_shape=jax.ShapeDtypeStruct((N_PAD, D), jnp.float32),
    )(x, w)


def _dis_of(dg_ref):
    return lax.rsqrt(dg_ref[:, 0:1] + 1.0)


def _scale_body(dg_ref, xw_ref, y_ref):
    y_ref[...] = _dis_of(dg_ref) * xw_ref[...]


def _scale(deg16, xw):
    return pl.pallas_call(
        _scale_body,
        grid=(GRID,),
        in_specs=[pl.BlockSpec((BLK, 16), lambda i: (i, 0)),
                  pl.BlockSpec((BLK, D), lambda i: (i, 0))],
        out_specs=pl.BlockSpec((BLK, D), lambda i: (i, 0)),
        out_shape=jax.ShapeDtypeStruct((N_PAD, D), jnp.float32),
    )(deg16, xw)


def _hrelu_body(s_ref, xw_ref, dg_ref, b1_ref, h_ref):
    dis = _dis_of(dg_ref)
    h_ref[...] = jnp.maximum(
        dis * s_ref[...] + dis * dis * xw_ref[...] + b1_ref[...], 0.0)


def _hrelu(S1, xw1, deg16, b1r):
    return pl.pallas_call(
        _hrelu_body,
        grid=(GRID,),
        in_specs=[pl.BlockSpec((BLK, D), lambda i: (i, 0)),
                  pl.BlockSpec((BLK, D), lambda i: (i, 0)),
                  pl.BlockSpec((BLK, 16), lambda i: (i, 0)),
                  pl.BlockSpec((1, D), lambda i: (0, 0))],
        out_specs=pl.BlockSpec((BLK, D), lambda i: (i, 0)),
        out_shape=jax.ShapeDtypeStruct((N_PAD, D), jnp.float32),
    )(S1, xw1, deg16, b1r)


def _mid_body(s_ref, xw_ref, dg_ref, w2_ref, b1_ref, xw2_ref, y2_ref):
    dis = _dis_of(dg_ref)
    h = jnp.maximum(dis * s_ref[...] + dis * dis * xw_ref[...] + b1_ref[...], 0.0)
    xw2 = jnp.dot(h, w2_ref[...], preferred_element_type=jnp.float32)
    xw2_ref[...] = xw2
    y2_ref[...] = dis * xw2


def _mid(S1, xw1, deg16, w2, b1r):
    return pl.pallas_call(
        _mid_body,
        grid=(GRID,),
        in_specs=[pl.BlockSpec((BLK, D), lambda i: (i, 0)),
                  pl.BlockSpec((BLK, D), lambda i: (i, 0)),
                  pl.BlockSpec((BLK, 16), lambda i: (i, 0)),
                  pl.BlockSpec((D, D), lambda i: (0, 0)),
                  pl.BlockSpec((1, D), lambda i: (0, 0))],
        out_specs=[pl.BlockSpec((BLK, D), lambda i: (i, 0)),
                   pl.BlockSpec((BLK, D), lambda i: (i, 0))],
        out_shape=[jax.ShapeDtypeStruct((N_PAD, D), jnp.float32),
                   jax.ShapeDtypeStruct((N_PAD, D), jnp.float32)],
    )(S1, xw1, deg16, w2, b1r)


def _final_body(s_ref, xw_ref, dg_ref, b2_ref, o_ref):
    dis = _dis_of(dg_ref)
    o_ref[...] = dis * s_ref[...] + dis * dis * xw_ref[...] + b2_ref[...]


def _final(S2, xw2, deg16, b2r):
    return pl.pallas_call(
        _final_body,
        grid=(GRID,),
        in_specs=[pl.BlockSpec((BLK, D), lambda i: (i, 0)),
                  pl.BlockSpec((BLK, D), lambda i: (i, 0)),
                  pl.BlockSpec((BLK, 16), lambda i: (i, 0)),
                  pl.BlockSpec((1, D), lambda i: (0, 0))],
        out_specs=pl.BlockSpec((BLK, D), lambda i: (i, 0)),
        out_shape=jax.ShapeDtypeStruct((N_PAD, D), jnp.float32),
    )(S2, xw2, deg16, b2r)


def kernel(x, edge_index, W1, b1, W2, b2):
    ei = edge_index.astype(jnp.int32)
    pad = E_PAD - E
    src3 = jnp.concatenate(
        [ei[0], jnp.full((pad,), DUMMY, jnp.int32)]).reshape(NW * NSLAB, SLAB, CHUNK)
    dst3 = jnp.concatenate(
        [ei[1], jnp.full((pad,), DUMMY, jnp.int32)]).reshape(NW * NSLAB, SLAB, CHUNK)
    x_pad = jnp.zeros((N_PAD, D), jnp.float32).at[:N_NODES, :].set(x)

    degp = _deg_kernel(dst3)
    degp, src3, dst3 = jax.lax.optimization_barrier((degp, src3, dst3))
    deg16 = degp[0] + degp[1]
    dis = jax.lax.rsqrt(deg16[:, 0:1] + 1.0)
    xw1 = _matmul(x_pad, W1)
    y1 = _scale(deg16, xw1)
    s1p = _prop_kernel(y1, src3, dst3)
    S1 = s1p[0] + s1p[1]
    h = _hrelu(S1, xw1, deg16, b1.reshape(1, D))
    xw2 = _matmul(h, W2)
    y2 = _scale(deg16, xw2)
    s2p = _prop_kernel(y2, src3, dst3)
    S2 = s2p[0] + s2p[1]
    out = _final(S2, xw2, deg16, b2.reshape(1, D))
    return out[:N_NODES]


# speedup vs baseline: 11.0904x; 1.2301x over previous
import functools

import jax
import jax.numpy as jnp
from jax import lax
from jax.experimental import pallas as pl
from jax.experimental.pallas import tpu as pltpu
from jax.experimental.pallas import tpu_sc as plsc

N_NODES = 10000
N_PAD = 10240
D = 128
E = 320000
NC, NS = 2, 16
NW = NC * NS
CHUNK = 128
N_CHUNKS = 80
E_PAD = NW * N_CHUNKS * CHUNK
SLAB = 20
NSLAB = N_CHUNKS // SLAB
SLABS_PER_STRIP = 2 * NSLAB
NSLAB_SLOW, NSLAB_FAST = 2, 6
DUMMY = N_NODES
ROWS_PER_TILE = N_PAD // NS
BLK = 512
GRID = N_PAD // BLK

_mesh = plsc.VectorSubcoreMesh(
    core_axis_name="c", subcore_axis_name="s", num_cores=NC, num_subcores=NS)


@functools.partial(
    pl.kernel,
    out_type=jax.ShapeDtypeStruct((NC, N_PAD, 16), jnp.float32),
    mesh=_mesh,
    scratch_types=[
        pltpu.VMEM((SLAB, CHUNK), jnp.int32),
        pltpu.VMEM((CHUNK,), jnp.int32),
        pltpu.VMEM((CHUNK, 16), jnp.float32),
        pltpu.VMEM((CHUNK, 16), jnp.float32),
        pltpu.VMEM_SHARED((N_PAD, 16), jnp.float32),
    ],
)
def _deg_kernel(dst_hbm, out_hbm, dst_v, dst_cur, ones_v, zeros_v, acc):
    c = lax.axis_index("c")
    s = lax.axis_index("s")
    wid = s * NC + c

    one16 = jnp.ones((16,), jnp.float32)
    zero16 = jnp.zeros((16,), jnp.float32)

    def _fill(r, _):
        ones_v[r, pl.ds(0, 16)] = one16
        zeros_v[r, pl.ds(0, 16)] = zero16
        return 0

    lax.fori_loop(0, CHUNK, _fill, 0)
    for k in range(ROWS_PER_TILE // CHUNK):
        pltpu.sync_copy(zeros_v, acc.at[pl.ds(s * ROWS_PER_TILE + k * CHUNK, CHUNK)])
    plsc.subcore_barrier()

    def _slab(t, _):
        pltpu.sync_copy(dst_hbm.at[wid * NSLAB + t], dst_v)

        def _body(j, _):
            for k in range(CHUNK // 16):
                dst_cur[pl.ds(k * 16, 16)] = dst_v[j, pl.ds(k * 16, 16)]
            pltpu.sync_copy(ones_v, acc.at[dst_cur], add=True)
            return 0

        lax.fori_loop(0, SLAB, _body, 0)
        return 0

    lax.fori_loop(0, NSLAB, _slab, 0)
    plsc.subcore_barrier()
    pltpu.sync_copy(acc.at[pl.ds(s * ROWS_PER_TILE, ROWS_PER_TILE)],
                    out_hbm.at[c, pl.ds(s * ROWS_PER_TILE, ROWS_PER_TILE)])


@functools.partial(
    pl.kernel,
    out_type=jax.ShapeDtypeStruct((NC, N_PAD, D), jnp.float32),
    mesh=_mesh,
    scratch_types=[
        pltpu.VMEM((SLAB, CHUNK), jnp.int32),
        pltpu.VMEM((SLAB, CHUNK), jnp.int32),
        pltpu.VMEM((CHUNK,), jnp.int32),
        pltpu.VMEM((CHUNK,), jnp.int32),
        pltpu.VMEM((CHUNK,), jnp.int32),
        pltpu.VMEM((CHUNK, D), jnp.float32),
        pltpu.VMEM((CHUNK, D), jnp.float32),
        pltpu.VMEM_SHARED((N_PAD, D), jnp.float32),
        pltpu.SemaphoreType.DMA,
        pltpu.SemaphoreType.DMA,
    ],
)
def _prop_kernel(y_hbm, src_hbm, dst_hbm, out_hbm,
                 src_v, dst_v, sidx0, sidx1, didx, buf0, buf1, acc, sem0, sem1):
    c = lax.axis_index("c")
    s = lax.axis_index("s")
    nslab_c = jnp.where(c == 0, NSLAB_FAST, NSLAB_SLOW)
    slab_base = s * SLABS_PER_STRIP + c * NSLAB_FAST

    zero16 = jnp.zeros((16,), jnp.float32)

    def _zrow(r, _):
        for k in range(D // 16):
            buf0[r, pl.ds(k * 16, 16)] = zero16
        return 0

    lax.fori_loop(0, CHUNK, _zrow, 0)
    for k in range(ROWS_PER_TILE // CHUNK):
        pltpu.sync_copy(buf0, acc.at[pl.ds(s * ROWS_PER_TILE + k * CHUNK, CHUNK)])
    plsc.subcore_barrier()

    def _fill(cur, row_ref, j):
        for k in range(CHUNK // 16):
            cur[pl.ds(k * 16, 16)] = row_ref[j, pl.ds(k * 16, 16)]

    def _slab(t, _):
        pltpu.sync_copy(src_hbm.at[slab_base + t], src_v)
        pltpu.sync_copy(dst_hbm.at[slab_base + t], dst_v)
        _fill(sidx0, src_v, 0)
        pltpu.async_copy(y_hbm.at[sidx0], buf0, sem0)
        _fill(sidx1, src_v, 1)
        pltpu.async_copy(y_hbm.at[sidx1], buf1, sem1)

        def _body(i, _):
            j = 2 * i
            pltpu.make_async_copy(y_hbm.at[sidx0], buf0, sem0).wait()
            _fill(didx, dst_v, j)
            pltpu.sync_copy(buf0, acc.at[didx], add=True)

            @pl.when(j + 2 < SLAB)
            def _():
                _fill(sidx0, src_v, j + 2)
                pltpu.async_copy(y_hbm.at[sidx0], buf0, sem0)

            pltpu.make_async_copy(y_hbm.at[sidx1], buf1, sem1).wait()
            _fill(didx, dst_v, j + 1)
            pltpu.sync_copy(buf1, acc.at[didx], add=True)

            @pl.when(j + 3 < SLAB)
            def _():
                _fill(sidx1, src_v, j + 3)
                pltpu.async_copy(y_hbm.at[sidx1], buf1, sem1)

            return 0

        lax.fori_loop(0, SLAB // 2, _body, 0)
        return 0

    lax.fori_loop(0, nslab_c, _slab, 0)
    plsc.subcore_barrier()
    pltpu.sync_copy(acc.at[pl.ds(s * ROWS_PER_TILE, ROWS_PER_TILE)],
                    out_hbm.at[c, pl.ds(s * ROWS_PER_TILE, ROWS_PER_TILE)])


def _mm_body(x_ref, w_ref, o_ref):
    o_ref[...] = jnp.dot(x_ref[...], w_ref[...],
                         preferred_element_type=jnp.float32)


def _matmul(x, w):
    return pl.pallas_call(
        _mm_body,
        grid=(GRID,),
        in_specs=[pl.BlockSpec((BLK, D), lambda i: (i, 0)),
                  pl.BlockSpec((D, D), lambda i: (0, 0))],
        out_specs=pl.BlockSpec((BLK, D), lambda i: (i, 0)),
        out_shape=jax.ShapeDtypeStruct((N_PAD, D), jnp.float32),
    )(x, w)


def _dis_of(dg_ref):
    return lax.rsqrt(dg_ref[:, 0:1] + 1.0)


def _scale_body(dg_ref, xw_ref, y_ref):
    y_ref[...] = _dis_of(dg_ref) * xw_ref[...]


def _scale(deg16, xw):
    return pl.pallas_call(
        _scale_body,
        grid=(GRID,),
        in_specs=[pl.BlockSpec((BLK, 16), lambda i: (i, 0)),
                  pl.BlockSpec((BLK, D), lambda i: (i, 0))],
        out_specs=pl.BlockSpec((BLK, D), lambda i: (i, 0)),
        out_shape=jax.ShapeDtypeStruct((N_PAD, D), jnp.float32),
    )(deg16, xw)


def _hrelu_body(s_ref, xw_ref, dg_ref, b1_ref, h_ref):
    dis = _dis_of(dg_ref)
    h_ref[...] = jnp.maximum(
        dis * s_ref[...] + dis * dis * xw_ref[...] + b1_ref[...], 0.0)


def _hrelu(S1, xw1, deg16, b1r):
    return pl.pallas_call(
        _hrelu_body,
        grid=(GRID,),
        in_specs=[pl.BlockSpec((BLK, D), lambda i: (i, 0)),
                  pl.BlockSpec((BLK, D), lambda i: (i, 0)),
                  pl.BlockSpec((BLK, 16), lambda i: (i, 0)),
                  pl.BlockSpec((1, D), lambda i: (0, 0))],
        out_specs=pl.BlockSpec((BLK, D), lambda i: (i, 0)),
        out_shape=jax.ShapeDtypeStruct((N_PAD, D), jnp.float32),
    )(S1, xw1, deg16, b1r)


def _mid_body(s_ref, xw_ref, dg_ref, w2_ref, b1_ref, xw2_ref, y2_ref):
    dis = _dis_of(dg_ref)
    h = jnp.maximum(dis * s_ref[...] + dis * dis * xw_ref[...] + b1_ref[...], 0.0)
    xw2 = jnp.dot(h, w2_ref[...], preferred_element_type=jnp.float32)
    xw2_ref[...] = xw2
    y2_ref[...] = dis * xw2


def _mid(S1, xw1, deg16, w2, b1r):
    return pl.pallas_call(
        _mid_body,
        grid=(GRID,),
        in_specs=[pl.BlockSpec((BLK, D), lambda i: (i, 0)),
                  pl.BlockSpec((BLK, D), lambda i: (i, 0)),
                  pl.BlockSpec((BLK, 16), lambda i: (i, 0)),
                  pl.BlockSpec((D, D), lambda i: (0, 0)),
                  pl.BlockSpec((1, D), lambda i: (0, 0))],
        out_specs=[pl.BlockSpec((BLK, D), lambda i: (i, 0)),
                   pl.BlockSpec((BLK, D), lambda i: (i, 0))],
        out_shape=[jax.ShapeDtypeStruct((N_PAD, D), jnp.float32),
                   jax.ShapeDtypeStruct((N_PAD, D), jnp.float32)],
    )(S1, xw1, deg16, w2, b1r)


def _final_body(s_ref, xw_ref, dg_ref, b2_ref, o_ref):
    dis = _dis_of(dg_ref)
    o_ref[...] = dis * s_ref[...] + dis * dis * xw_ref[...] + b2_ref[...]


def _final(S2, xw2, deg16, b2r):
    return pl.pallas_call(
        _final_body,
        grid=(GRID,),
        in_specs=[pl.BlockSpec((BLK, D), lambda i: (i, 0)),
                  pl.BlockSpec((BLK, D), lambda i: (i, 0)),
                  pl.BlockSpec((BLK, 16), lambda i: (i, 0)),
                  pl.BlockSpec((1, D), lambda i: (0, 0))],
        out_specs=pl.BlockSpec((BLK, D), lambda i: (i, 0)),
        out_shape=jax.ShapeDtypeStruct((N_PAD, D), jnp.float32),
    )(S2, xw2, deg16, b2r)


def kernel(x, edge_index, W1, b1, W2, b2):
    ei = edge_index.astype(jnp.int32)
    pad = E_PAD - E
    src3 = jnp.concatenate(
        [ei[0], jnp.full((pad,), DUMMY, jnp.int32)]).reshape(NW * NSLAB, SLAB, CHUNK)
    dst3 = jnp.concatenate(
        [ei[1], jnp.full((pad,), DUMMY, jnp.int32)]).reshape(NW * NSLAB, SLAB, CHUNK)
    x_pad = jnp.zeros((N_PAD, D), jnp.float32).at[:N_NODES, :].set(x)

    degp = _deg_kernel(dst3)
    degp, src3, dst3 = jax.lax.optimization_barrier((degp, src3, dst3))
    deg16 = degp[0] + degp[1]
    dis = jax.lax.rsqrt(deg16[:, 0:1] + 1.0)
    xw1 = _matmul(x_pad, W1)
    y1 = _scale(deg16, xw1)
    s1p = _prop_kernel(y1, src3, dst3)
    S1 = s1p[0] + s1p[1]
    h = _hrelu(S1, xw1, deg16, b1.reshape(1, D))
    xw2 = _matmul(h, W2)
    y2 = _scale(deg16, xw2)
    s2p = _prop_kernel(y2, src3, dst3)
    S2 = s2p[0] + s2p[1]
    out = _final(S2, xw2, deg16, b2.reshape(1, D))
    return out[:N_NODES]
